# baseline (speedup 1.0000x reference)
import functools

import jax
import jax.numpy as jnp
from jax import lax
from jax.experimental import pallas as pl
from jax.experimental.pallas import tpu as pltpu

N = 10000
E = 160000
ETOT = E + N
IN = 256
HID = 256
HEADS = 2
NREL = 16

BR = 512
NPAD = 10240


def _gelu(x):
    return 0.5 * x * (1.0 + lax.erf(x * 0.7071067811865476))


def _tc1_body(hp_ref, agg_ref, wg_ref, bg_ref, w1_ref, b1_ref, h1_ref, f1_ref):
    x = hp_ref[...] + agg_ref[...]
    h1 = _gelu(jnp.dot(x, wg_ref[...], preferred_element_type=jnp.float32)
               + bg_ref[...])
    h1_ref[...] = h1
    f1_ref[...] = jnp.dot(h1, w1_ref[...], preferred_element_type=jnp.float32) + b1_ref[...]


def _tc1(h_prev, agg, wg, bg, w1, b1):
    k = w1.shape[1]
    return pl.pallas_call(
        _tc1_body,
        grid=(NPAD // BR,),
        in_specs=[
            pl.BlockSpec((BR, IN), lambda i: (i, 0)),
            pl.BlockSpec((BR, IN), lambda i: (i, 0)),
            pl.BlockSpec((IN, HID), lambda i: (0, 0)),
            pl.BlockSpec((HID,), lambda i: (0,)),
            pl.BlockSpec((HID, k), lambda i: (0, 0)),
            pl.BlockSpec((k,), lambda i: (0,)),
        ],
        out_specs=[
            pl.BlockSpec((BR, HID), lambda i: (i, 0)),
            pl.BlockSpec((BR, k), lambda i: (i, 0)),
        ],
        out_shape=[
            jax.ShapeDtypeStruct((NPAD, HID), jnp.float32),
            jax.ShapeDtypeStruct((NPAD, k), jnp.float32),
        ],
    )(h_prev, agg, wg, bg, w1, b1)


def _tc2_body(o_ref, w2_ref, b2_ref, f2_ref):
    o = o_ref[...]
    d = o.shape[1] // 2
    h2 = _gelu(o[:, :d]) + _gelu(o[:, d:])
    f2_ref[...] = jnp.dot(h2, w2_ref[...], preferred_element_type=jnp.float32) + b2_ref[...]


def _tc2(o, w2, b2):
    d2 = o.shape[1]
    k = w2.shape[1]
    return pl.pallas_call(
        _tc2_body,
        grid=(NPAD // BR,),
        in_specs=[
            pl.BlockSpec((BR, d2), lambda i: (i, 0)),
            pl.BlockSpec((d2 // 2, k), lambda i: (0, 0)),
            pl.BlockSpec((k,), lambda i: (0,)),
        ],
        out_specs=pl.BlockSpec((BR, k), lambda i: (i, 0)),
        out_shape=jax.ShapeDtypeStruct((NPAD, k), jnp.float32),
    )(o, w2, b2)


def _tc3_body(o_ref, h_ref):
    o = o_ref[...]
    d = o.shape[1] // 2
    h_ref[...] = _gelu(o[:, :d]) + _gelu(o[:, d:])


def _tc3(o):
    d2 = o.shape[1]
    return pl.pallas_call(
        _tc3_body,
        grid=(NPAD // BR,),
        in_specs=[pl.BlockSpec((BR, d2), lambda i: (i, 0))],
        out_specs=pl.BlockSpec((BR, d2 // 2), lambda i: (i, 0)),
        out_shape=jax.ShapeDtypeStruct((NPAD, d2 // 2), jnp.float32),
    )(o)


def _gine_agg(h_prev, relation_memory, src, dst, etype):
    e_feat = jnp.concatenate(
        [relation_memory[etype], jnp.zeros((N, IN), dtype=jnp.float32)], axis=0)
    msg = jax.nn.relu(h_prev[src] + e_feat)
    return jax.ops.segment_sum(msg, dst, num_segments=N)


def _gat_edge(f, src, dst, attn, out_dim):
    fs = f[:, :HEADS * out_dim].reshape(N, HEADS, out_dim)
    fd = f[:, HEADS * out_dim:].reshape(N, HEADS, out_dim)
    e = jax.nn.leaky_relu(fs[src] + fd[dst], negative_slope=0.2)
    score = jnp.einsum("ehd,hd->eh", e, attn)
    smax = jax.ops.segment_max(score, dst, num_segments=N)
    smax = jnp.where(jnp.isfinite(smax), smax, 0.0)
    ex = jnp.exp(score - smax[dst])
    denom = jax.ops.segment_sum(ex, dst, num_segments=N)
    alpha = ex / jnp.maximum(denom[dst], 1e-9)
    out = jax.ops.segment_sum(alpha[..., None] * fs[src], dst, num_segments=N)
    return out.reshape(N, HEADS * out_dim)


def kernel(h_prev, relation_memory, edge_index, etype, node_indices,
           W_gine, b_gine, W1_src, b1_src, W1_dst, b1_dst, attn1,
           W2_src, b2_src, W2_dst, b2_dst, attn2):
    loop = jnp.arange(N, dtype=edge_index.dtype)
    src = jnp.concatenate([edge_index[0], loop])
    dst = jnp.concatenate([edge_index[1], loop])

    agg = _gine_agg(h_prev, relation_memory, src, dst, etype)

    hp_pad = jnp.zeros((NPAD, IN), jnp.float32).at[:N].set(h_prev)
    agg_pad = jnp.zeros((NPAD, IN), jnp.float32).at[:N].set(agg)

    w1 = jnp.concatenate([W1_src, W1_dst], axis=1)
    b1 = jnp.concatenate([b1_src, b1_dst], axis=0)
    _, f1 = _tc1(hp_pad, agg_pad, W_gine, b_gine, w1, b1)
    f1 = f1[:N]

    o1 = _gat_edge(f1, src, dst, attn1, HID)
    o1_pad = jnp.zeros((NPAD, HEADS * HID), jnp.float32).at[:N].set(o1)

    w2 = jnp.concatenate([W2_src, W2_dst], axis=1)
    b2 = jnp.concatenate([b2_src, b2_dst], axis=0)
    f2 = _tc2(o1_pad, w2, b2)[:N]

    o2 = _gat_edge(f2, src, dst, attn2, IN)
    o2_pad = jnp.zeros((NPAD, HEADS * IN), jnp.float32).at[:N].set(o2)
    h = _tc3(o2_pad)[:N]

    n_pes = node_indices.shape[0]
    return jnp.concatenate([h[:n_pes], h[:n_pes], h[n_pes:], h[n_pes:]], axis=0)
